# Initial kernel scaffold; baseline (speedup 1.0000x reference)
import jax
import jax.numpy as jnp
from jax.experimental import pallas as pl


def kernel(rois, classifications):
    raise NotImplementedError("write your pallas kernel here")


# speedup vs baseline: 9.6795x; 9.6795x over previous
import jax
import jax.numpy as jnp
from jax.experimental import pallas as pl

_B = 4
_N = 5000
_NPAD = 5120
_MAXDET = 100
_MINCONF = 0.7
_NMS_T = 0.3


def _nms_kernel(rois_ref, cls_ref, out_ref):
    y1 = rois_ref[:, 0, :]
    x1 = rois_ref[:, 1, :]
    y2 = rois_ref[:, 2, :]
    x2 = rois_ref[:, 3, :]
    dy = cls_ref[:, 0, :] * 0.1
    dx = cls_ref[:, 1, :] * 0.1
    dh = cls_ref[:, 2, :] * 0.2
    dw = cls_ref[:, 3, :] * 0.2
    cls_f = cls_ref[:, 4, :]
    raw_scores = cls_ref[:, 5, :]

    h = y2 - y1
    w = x2 - x1
    cy = y1 + 0.5 * h + dy * h
    cx = x1 + 0.5 * w + dx * w
    h = h * jnp.exp(dh)
    w = w * jnp.exp(dw)
    ry1 = cy - 0.5 * h
    rx1 = cx - 0.5 * w
    ry2 = ry1 + h
    rx2 = rx1 + w
    ry1 = jnp.clip(ry1, 0.0, 1.0)
    rx1 = jnp.clip(rx1, 0.0, 1.0)
    ry2 = jnp.clip(ry2, 0.0, 1.0)
    rx2 = jnp.clip(rx2, 0.0, 1.0)

    cls_i = cls_f.astype(jnp.int32)
    keep = (cls_i > 0) & (raw_scores >= _MINCONF)
    scores0 = jnp.where(keep, raw_scores, -1.0)

    off = cls_f * 4.0
    ny1 = ry1 + off
    nx1 = rx1 + off
    ny2 = ry2 + off
    nx2 = rx2 + off
    areas = (ny2 - ny1) * (nx2 - nx1)

    iota = jax.lax.broadcasted_iota(jnp.int32, (_B, _NPAD), 1)
    big = jnp.int32(_NPAD + 1)
    lane = jax.lax.broadcasted_iota(jnp.int32, (_B, 6, 128), 2)

    def body(i, carry):
        scores, acc = carry
        best = jnp.max(scores, axis=1, keepdims=True)
        ismax = scores == best
        idx = jnp.min(jnp.where(ismax, iota, big), axis=1, keepdims=True)
        isbest = iota == idx
        zero = jnp.zeros_like(scores)

        def sel(v):
            return jnp.sum(jnp.where(isbest, v, zero), axis=1, keepdims=True)

        by1 = sel(ny1)
        bx1 = sel(nx1)
        by2 = sel(ny2)
        bx2 = sel(nx2)
        barea = sel(areas)
        valid = best > 0.0

        yy1 = jnp.maximum(by1, ny1)
        xx1 = jnp.maximum(bx1, nx1)
        yy2 = jnp.minimum(by2, ny2)
        xx2 = jnp.minimum(bx2, nx2)
        inter = jnp.maximum(yy2 - yy1, 0.0) * jnp.maximum(xx2 - xx1, 0.0)
        iou = inter / (barea + areas - inter + 1e-8)
        suppress = (iou > _NMS_T) | isbest
        new_scores = jnp.where(valid & suppress, -1.0, scores)

        vmask = valid[:, 0]
        det = jnp.concatenate(
            [
                sel(ry1),
                sel(rx1),
                sel(ry2),
                sel(rx2),
                sel(cls_f),
                sel(raw_scores),
            ],
            axis=1,
        )
        det = jnp.where(vmask[:, None], det, 0.0)
        acc = acc + jnp.where(lane == i, det[:, :, None], 0.0)
        return new_scores, acc

    acc0 = jnp.zeros((_B, 6, 128), jnp.float32)
    _, acc = jax.lax.fori_loop(0, _MAXDET, body, (scores0, acc0))
    out_ref[...] = acc


def kernel(rois, classifications):
    rois_t = jnp.transpose(rois, (0, 2, 1))
    cls_t = jnp.transpose(classifications, (0, 2, 1))
    pad = _NPAD - _N
    rois_t = jnp.pad(rois_t, ((0, 0), (0, 0), (0, pad)))
    cls_t = jnp.pad(cls_t, ((0, 0), (0, 0), (0, pad)))

    out = pl.pallas_call(
        _nms_kernel,
        out_shape=jax.ShapeDtypeStruct((_B, 6, 128), jnp.float32),
    )(rois_t, cls_t)
    return jnp.transpose(out[:, :, :_MAXDET], (0, 2, 1))
